# baseline (speedup 1.0000x reference)
import jax
import jax.numpy as jnp
from jax import lax
from jax.experimental import pallas as pl
from jax.experimental.pallas import tpu as pltpu
from jax.experimental.pallas import tpu_sc as plsc

N_FEAT = 26
VOCAB = 120
EMB = 32
BATCH = 16384
HID = 256
OUT_DIM = 64
PK = EMB // 2
XW = N_FEAT * PK
TROW = PK + 1
TILE_B = 1024

NC = 2
NS = 16
NW = NC * NS
ROWS_PER = BATCH // NW
CHUNK = 128
NCHUNK = ROWS_PER // CHUNK
NPLANE = BATCH // CHUNK
TAB_WORDS = VOCAB * TROW
IDX_PER_W = N_FEAT * ROWS_PER


HALF = N_FEAT // 2
HROWS = HALF * PK


def _gather_body(*args):
    frefs = args[:N_FEAT]
    tab_hbm, x_hbm, idx_v, tab_v, xt_v, sem_a, sem_b, sem_i = args[N_FEAT:]
    wid = lax.axis_index("s") * NC + lax.axis_index("c")
    pltpu.sync_copy(tab_hbm, tab_v)
    for j in range(N_FEAT):
        pltpu.async_copy(frefs[j].at[pl.ds(wid * ROWS_PER, ROWS_PER)],
                         idx_v.at[pl.ds(j * ROWS_PER, ROWS_PER)], sem_i)
    for j in range(N_FEAT):
        pltpu.make_async_copy(
            frefs[0].at[pl.ds(0, ROWS_PER)],
            idx_v.at[pl.ds(j * ROWS_PER, ROWS_PER)], sem_i).wait()

    @pl.loop(0, NCHUNK)
    def _chunk(cl):
        gc = wid * NCHUNK + cl

        def fill(j0, j1):
            @pl.loop(j0, j1, unroll=2)
            def _feat(j):
                for g in range(CHUNK // 16):
                    fv = idx_v[pl.ds(j * ROWS_PER + cl * CHUNK + g * 16, 16)]
                    tbase = fv * TROW + j * TAB_WORDS
                    ws = {}
                    for cp in range(PK):
                        ws[cp] = plsc.load_gather(tab_v, [tbase + cp])
                        if cp >= 4:
                            xt_v[j * PK + cp - 4, pl.ds(g * 16, 16)] = \
                                ws.pop(cp - 4)
                    for cp in range(PK - 4, PK):
                        xt_v[j * PK + cp, pl.ds(g * 16, 16)] = ws.pop(cp)

        def drain(r0, sem):
            pltpu.make_async_copy(x_hbm.at[0, pl.ds(r0, HROWS)],
                                  xt_v.at[pl.ds(r0, HROWS)], sem).wait()

        @pl.when(cl > 0)
        def _():
            drain(0, sem_a)
        fill(0, HALF)
        pltpu.async_copy(xt_v.at[pl.ds(0, HROWS)],
                         x_hbm.at[gc, pl.ds(0, HROWS)], sem_a)

        @pl.when(cl > 0)
        def _():
            drain(HROWS, sem_b)
        fill(HALF, N_FEAT)
        pltpu.async_copy(xt_v.at[pl.ds(HROWS, HROWS)],
                         x_hbm.at[gc, pl.ds(HROWS, HROWS)], sem_b)

    pltpu.make_async_copy(x_hbm.at[0, pl.ds(0, HROWS)],
                          xt_v.at[pl.ds(0, HROWS)], sem_a).wait()
    pltpu.make_async_copy(x_hbm.at[0, pl.ds(HROWS, HROWS)],
                          xt_v.at[pl.ds(HROWS, HROWS)], sem_b).wait()


def _mlp_body(x_ref, w1e_ref, w1o_ref, b1_ref, w2_ref, b2_ref, out_ref):
    hs = []
    for c in range(TILE_B // CHUNK):
        xw = x_ref[c]
        xe = lax.bitcast_convert_type(xw << 16, jnp.float32)
        xo = lax.bitcast_convert_type(xw & jnp.int32(-65536), jnp.float32)
        dn = (((0,), (0,)), ((), ()))
        hc = lax.dot_general(xe.astype(jnp.bfloat16), w1e_ref[...], dn,
                             preferred_element_type=jnp.float32)
        hc = hc + lax.dot_general(xo.astype(jnp.bfloat16), w1o_ref[...], dn,
                                  preferred_element_type=jnp.float32)
        hs.append(hc)
    h = jnp.concatenate(hs, axis=0)
    h = jnp.maximum(h + b1_ref[...], 0.0)
    z = jax.lax.dot(h, w2_ref[...], preferred_element_type=jnp.float32)
    z = z + b2_ref[...]
    n = jnp.sqrt(jnp.sum(z * z, axis=1, keepdims=True))
    out_ref[...] = z / jnp.maximum(n, 1e-12)


def kernel(f0, emb_f0, f1, emb_f1, f2, emb_f2, f3, emb_f3, f4, emb_f4,
           f5, emb_f5, f6, emb_f6, f7, emb_f7, f8, emb_f8, f9, emb_f9,
           f10, emb_f10, f11, emb_f11, f12, emb_f12, f13, emb_f13,
           f14, emb_f14, f15, emb_f15, f16, emb_f16, f17, emb_f17,
           f18, emb_f18, f19, emb_f19, f20, emb_f20, f21, emb_f21,
           f22, emb_f22, f23, emb_f23, f24, emb_f24, f25, emb_f25,
           W1, b1, W2, b2):
    feats = [f0, f1, f2, f3, f4, f5, f6, f7, f8, f9, f10, f11, f12,
             f13, f14, f15, f16, f17, f18, f19, f20, f21, f22, f23,
             f24, f25]
    tabs = [emb_f0, emb_f1, emb_f2, emb_f3, emb_f4, emb_f5, emb_f6,
            emb_f7, emb_f8, emb_f9, emb_f10, emb_f11, emb_f12, emb_f13,
            emb_f14, emb_f15, emb_f16, emb_f17, emb_f18, emb_f19,
            emb_f20, emb_f21, emb_f22, emb_f23, emb_f24, emb_f25]
    tab = lax.optimization_barrier(jnp.stack(tabs, axis=0))
    tab = tab.astype(jnp.bfloat16)
    tab = lax.bitcast_convert_type(tab.reshape(N_FEAT, VOCAB, PK, 2),
                                   jnp.int32)
    tab = jnp.pad(tab, ((0, 0), (0, 0), (0, 1))).reshape(-1)

    gather = pl.kernel(
        _gather_body,
        out_type=jax.ShapeDtypeStruct((NPLANE, XW, CHUNK), jnp.int32),
        mesh=plsc.VectorSubcoreMesh(core_axis_name="c", subcore_axis_name="s"),
        compiler_params=pltpu.CompilerParams(needs_layout_passes=False),
        scratch_types=[
            pltpu.VMEM((IDX_PER_W,), jnp.int32),
            pltpu.VMEM((N_FEAT * TAB_WORDS,), jnp.int32),
            pltpu.VMEM((XW, CHUNK), jnp.int32),
            pltpu.SemaphoreType.DMA,
            pltpu.SemaphoreType.DMA,
            pltpu.SemaphoreType.DMA,
        ],
    )
    x = gather(*feats, tab)
    w1b = lax.optimization_barrier(W1.astype(jnp.bfloat16))
    w1lo = w1b[0::2]
    w1hi = w1b[1::2]

    grid = BATCH // TILE_B
    return pl.pallas_call(
        _mlp_body,
        grid=(grid,),
        in_specs=[
            pl.BlockSpec((TILE_B // CHUNK, XW, CHUNK), lambda i: (i, 0, 0)),
            pl.BlockSpec((XW, HID), lambda i: (0, 0)),
            pl.BlockSpec((XW, HID), lambda i: (0, 0)),
            pl.BlockSpec((HID,), lambda i: (0,)),
            pl.BlockSpec((HID, OUT_DIM), lambda i: (0, 0)),
            pl.BlockSpec((OUT_DIM,), lambda i: (0,)),
        ],
        out_specs=pl.BlockSpec((TILE_B, OUT_DIM), lambda i: (i, 0)),
        out_shape=jax.ShapeDtypeStruct((BATCH, OUT_DIM), jnp.float32),
    )(x, w1lo, w1hi, b1, W2, b2)


# speedup vs baseline: 1.0124x; 1.0124x over previous
import jax
import jax.numpy as jnp
from jax import lax
from jax.experimental import pallas as pl
from jax.experimental.pallas import tpu as pltpu
from jax.experimental.pallas import tpu_sc as plsc

N_FEAT = 26
VOCAB = 120
EMB = 32
BATCH = 16384
HID = 256
OUT_DIM = 64
PK = EMB // 2
XW = N_FEAT * PK
TROW = PK + 1
TILE_B = 1024

NC = 2
NS = 16
NW = NC * NS
ROWS_PER = BATCH // NW
CHUNK = 128
NCHUNK = ROWS_PER // CHUNK
NPLANE = BATCH // CHUNK
TAB_WORDS = VOCAB * TROW
IDX_PER_W = N_FEAT * ROWS_PER


HALF = N_FEAT // 2
HROWS = HALF * PK


def _gather_body(*args):
    frefs = args[:N_FEAT]
    tab_hbm, x_hbm, idx_v, tab_v, xt_v, sem_a, sem_b, sem_i = args[N_FEAT:]
    wid = lax.axis_index("s") * NC + lax.axis_index("c")
    pltpu.sync_copy(tab_hbm, tab_v)
    for j in range(N_FEAT):
        pltpu.async_copy(frefs[j].at[pl.ds(wid * ROWS_PER, ROWS_PER)],
                         idx_v.at[pl.ds(j * ROWS_PER, ROWS_PER)], sem_i)
    for j in range(N_FEAT):
        pltpu.make_async_copy(
            frefs[0].at[pl.ds(0, ROWS_PER)],
            idx_v.at[pl.ds(j * ROWS_PER, ROWS_PER)], sem_i).wait()

    @pl.loop(0, NCHUNK)
    def _chunk(cl):
        gc = wid * NCHUNK + cl

        def fill(j0, j1):
            @pl.loop(j0, j1)
            def _feat(j):
                for g in range(CHUNK // 16):
                    fv = idx_v[pl.ds(j * ROWS_PER + cl * CHUNK + g * 16, 16)]
                    tbase = fv * TROW + j * TAB_WORDS
                    ws = [plsc.load_gather(tab_v, [tbase + cp])
                          for cp in range(PK)]
                    for cp in range(PK):
                        xt_v[j * PK + cp, pl.ds(g * 16, 16)] = ws[cp]

        def drain(r0, sem):
            pltpu.make_async_copy(x_hbm.at[0, pl.ds(r0, HROWS)],
                                  xt_v.at[pl.ds(r0, HROWS)], sem).wait()

        @pl.when(cl > 0)
        def _():
            drain(0, sem_a)
        fill(0, HALF)
        pltpu.async_copy(xt_v.at[pl.ds(0, HROWS)],
                         x_hbm.at[gc, pl.ds(0, HROWS)], sem_a)

        @pl.when(cl > 0)
        def _():
            drain(HROWS, sem_b)
        fill(HALF, N_FEAT)
        pltpu.async_copy(xt_v.at[pl.ds(HROWS, HROWS)],
                         x_hbm.at[gc, pl.ds(HROWS, HROWS)], sem_b)

    pltpu.make_async_copy(x_hbm.at[0, pl.ds(0, HROWS)],
                          xt_v.at[pl.ds(0, HROWS)], sem_a).wait()
    pltpu.make_async_copy(x_hbm.at[0, pl.ds(HROWS, HROWS)],
                          xt_v.at[pl.ds(HROWS, HROWS)], sem_b).wait()


def _mlp_body(x_ref, w1e_ref, w1o_ref, b1_ref, w2_ref, b2_ref, out_ref):
    hs = []
    for c in range(TILE_B // CHUNK):
        xw = x_ref[c]
        xe = lax.bitcast_convert_type(xw << 16, jnp.float32)
        xo = lax.bitcast_convert_type(xw & jnp.int32(-65536), jnp.float32)
        dn = (((0,), (0,)), ((), ()))
        hc = lax.dot_general(xe.astype(jnp.bfloat16), w1e_ref[...], dn,
                             preferred_element_type=jnp.float32)
        hc = hc + lax.dot_general(xo.astype(jnp.bfloat16), w1o_ref[...], dn,
                                  preferred_element_type=jnp.float32)
        hs.append(hc)
    h = jnp.concatenate(hs, axis=0)
    h = jnp.maximum(h + b1_ref[...], 0.0)
    z = jax.lax.dot(h, w2_ref[...], preferred_element_type=jnp.float32)
    z = z + b2_ref[...]
    n = jnp.sqrt(jnp.sum(z * z, axis=1, keepdims=True))
    out_ref[...] = z / jnp.maximum(n, 1e-12)


def kernel(f0, emb_f0, f1, emb_f1, f2, emb_f2, f3, emb_f3, f4, emb_f4,
           f5, emb_f5, f6, emb_f6, f7, emb_f7, f8, emb_f8, f9, emb_f9,
           f10, emb_f10, f11, emb_f11, f12, emb_f12, f13, emb_f13,
           f14, emb_f14, f15, emb_f15, f16, emb_f16, f17, emb_f17,
           f18, emb_f18, f19, emb_f19, f20, emb_f20, f21, emb_f21,
           f22, emb_f22, f23, emb_f23, f24, emb_f24, f25, emb_f25,
           W1, b1, W2, b2):
    feats = [f0, f1, f2, f3, f4, f5, f6, f7, f8, f9, f10, f11, f12,
             f13, f14, f15, f16, f17, f18, f19, f20, f21, f22, f23,
             f24, f25]
    tabs = [emb_f0, emb_f1, emb_f2, emb_f3, emb_f4, emb_f5, emb_f6,
            emb_f7, emb_f8, emb_f9, emb_f10, emb_f11, emb_f12, emb_f13,
            emb_f14, emb_f15, emb_f16, emb_f17, emb_f18, emb_f19,
            emb_f20, emb_f21, emb_f22, emb_f23, emb_f24, emb_f25]
    tab = lax.optimization_barrier(jnp.stack(tabs, axis=0))
    tab = tab.astype(jnp.bfloat16)
    tab = lax.bitcast_convert_type(tab.reshape(N_FEAT, VOCAB, PK, 2),
                                   jnp.int32)
    tab = jnp.pad(tab, ((0, 0), (0, 0), (0, 1))).reshape(-1)

    gather = pl.kernel(
        _gather_body,
        out_type=jax.ShapeDtypeStruct((NPLANE, XW, CHUNK), jnp.int32),
        mesh=plsc.VectorSubcoreMesh(core_axis_name="c", subcore_axis_name="s"),
        compiler_params=pltpu.CompilerParams(needs_layout_passes=False),
        scratch_types=[
            pltpu.VMEM((IDX_PER_W,), jnp.int32),
            pltpu.VMEM((N_FEAT * TAB_WORDS,), jnp.int32),
            pltpu.VMEM((XW, CHUNK), jnp.int32),
            pltpu.SemaphoreType.DMA,
            pltpu.SemaphoreType.DMA,
            pltpu.SemaphoreType.DMA,
        ],
    )
    x = gather(*feats, tab)
    w1b = lax.optimization_barrier(W1.astype(jnp.bfloat16))
    w1lo = w1b[0::2]
    w1hi = w1b[1::2]

    grid = BATCH // TILE_B
    return pl.pallas_call(
        _mlp_body,
        grid=(grid,),
        in_specs=[
            pl.BlockSpec((TILE_B // CHUNK, XW, CHUNK), lambda i: (i, 0, 0)),
            pl.BlockSpec((XW, HID), lambda i: (0, 0)),
            pl.BlockSpec((XW, HID), lambda i: (0, 0)),
            pl.BlockSpec((HID,), lambda i: (0,)),
            pl.BlockSpec((HID, OUT_DIM), lambda i: (0, 0)),
            pl.BlockSpec((OUT_DIM,), lambda i: (0,)),
        ],
        out_specs=pl.BlockSpec((TILE_B, OUT_DIM), lambda i: (i, 0)),
        out_shape=jax.ShapeDtypeStruct((BATCH, OUT_DIM), jnp.float32),
    )(x, w1lo, w1hi, b1, W2, b2)
